# baseline (speedup 1.0000x reference)
import functools

import jax
import jax.numpy as jnp
from jax import lax
from jax.experimental import pallas as pl
from jax.experimental.pallas import tpu as pltpu
from jax.experimental.pallas import tpu_sc as plsc

_NC = 2
_NS = 16
_NW = _NC * _NS
_CK = 128
_G = 4


def _make_sc_gather(V, D, N):
    assert N % (_NW * _CK) == 0
    ch = N // (_NW * _CK)
    assert ch % _G == 0
    ng = ch // _G
    mesh = plsc.VectorSubcoreMesh(core_axis_name="c", subcore_axis_name="s")

    @functools.partial(
        pl.kernel,
        out_type=jax.ShapeDtypeStruct((N, D), jnp.float32),
        mesh=mesh,
        scratch_types=[
            pltpu.VMEM((ch, _CK), jnp.int32),
            pltpu.VMEM((_G * _CK, D), jnp.float32),
            pltpu.SemaphoreType.DMA,
        ],
        compiler_params=pltpu.CompilerParams(use_tc_tiling_on_sc=False),
    )
    def sc_gather(table_hbm, idx_hbm, out_hbm, idx_v, rows_v, gsem):
        cid = lax.axis_index("c")
        sid = lax.axis_index("s")
        wid = cid * _NS + sid
        base = wid * ch * _CK
        pltpu.sync_copy(idx_hbm.at[wid], idx_v)

        def group(g, carry):
            cps = []
            for k in range(_G):
                cps.append(
                    pltpu.async_copy(
                        table_hbm.at[idx_v.at[g * _G + k]],
                        rows_v.at[pl.ds(k * _CK, _CK)],
                        gsem,
                    )
                )
            for cp in cps:
                cp.wait()
            pltpu.sync_copy(
                rows_v,
                out_hbm.at[pl.ds(base + g * (_G * _CK), _G * _CK)],
            )
            return carry

        lax.fori_loop(0, ng, group, 0)

    return sc_gather


def _make_tc_linear(N, D, bs):
    assert N % bs == 0

    def body(x_ref, wt_ref, b_ref, o_ref):
        y = jnp.dot(x_ref[...], wt_ref[...], preferred_element_type=jnp.float32)
        o_ref[...] = jnp.maximum(y + b_ref[...], 0.0)

    return pl.pallas_call(
        body,
        grid=(N // bs,),
        in_specs=[
            pl.BlockSpec((bs, D), lambda i: (i, 0)),
            pl.BlockSpec((D, D), lambda i: (0, 0)),
            pl.BlockSpec((1, D), lambda i: (0, 0)),
        ],
        out_specs=pl.BlockSpec((bs, D), lambda i: (i, 0)),
        out_shape=jax.ShapeDtypeStruct((N, D), jnp.float32),
    )


def kernel(category, table, W, b):
    B, F = category.shape
    V, D = table.shape
    N = B * F

    ch = N // (_NW * _CK)
    idx3 = category.reshape(_NW, ch, _CK)

    gathered = _make_sc_gather(V, D, N)(table, idx3)
    out = _make_tc_linear(N, D, 2048)(gathered, W.T, b.reshape(1, D))
    return out.reshape(B, F, D)


# speedup vs baseline: 2.3444x; 2.3444x over previous
import functools

import jax
import jax.numpy as jnp
from jax import lax
from jax.experimental import pallas as pl
from jax.experimental.pallas import tpu as pltpu
from jax.experimental.pallas import tpu_sc as plsc

_NC = 2
_NS = 16
_NW = _NC * _NS
_CK = 128
_G = 4


_SB = 8192


def _make_transform(V, D):
    nblk = -(-V // _SB)
    HB = _SB // 2

    def body(x_ref, w_ref, b_ref, o_ref):
        dn = (((0,), (1,)), ((), ()))
        x = x_ref[...]
        y1 = lax.dot_general(x[:, :HB], w_ref[...], dn,
                             preferred_element_type=jnp.float32)
        y2 = lax.dot_general(x[:, HB:], w_ref[...], dn,
                             preferred_element_type=jnp.float32)
        y1 = jnp.maximum(y1 + b_ref[...], 0.0)
        y2 = jnp.maximum(y2 + b_ref[...], 0.0)
        o_ref[...] = jnp.concatenate([y1, y2], axis=1)

    return pl.pallas_call(
        body,
        grid=(nblk,),
        in_specs=[
            pl.BlockSpec((D, _SB), lambda i: (0, i)),
            pl.BlockSpec((D, D), lambda i: (0, 0)),
            pl.BlockSpec((1, D), lambda i: (0, 0)),
        ],
        out_specs=pl.BlockSpec((HB, 2 * D), lambda i: (i, 0)),
        out_shape=jax.ShapeDtypeStruct((nblk * HB, 2 * D), jnp.float32),
    )


def _make_sc_gather(VL, D, N):
    assert N % (_NW * _CK) == 0
    ch = N // (_NW * _CK)
    assert ch % _G == 0
    ng = ch // _G
    mesh = plsc.VectorSubcoreMesh(core_axis_name="c", subcore_axis_name="s")

    @functools.partial(
        pl.kernel,
        out_type=jax.ShapeDtypeStruct((N, D), jnp.float32),
        mesh=mesh,
        scratch_types=[
            pltpu.VMEM((ch, _CK), jnp.int32),
            pltpu.VMEM((_G * _CK, D), jnp.float32),
            pltpu.SemaphoreType.DMA,
        ],
        compiler_params=pltpu.CompilerParams(use_tc_tiling_on_sc=False),
    )
    def sc_gather(table_hbm, idx_hbm, out_hbm, idx_v, rows_v, gsem):
        cid = lax.axis_index("c")
        sid = lax.axis_index("s")
        wid = cid * _NS + sid
        base = wid * ch * _CK
        pltpu.sync_copy(idx_hbm.at[wid], idx_v)

        def xform(r, carry):
            for k in range(_CK // 16):
                t = idx_v[r, pl.ds(k * 16, 16)]
                a = t & (-_SB)
                m = t & (_SB // 2 - 1)
                h = (t >> (13 - 1)) & 1
                idx_v[r, pl.ds(k * 16, 16)] = a + m + m + h
            return carry

        lax.fori_loop(0, ch, xform, 0)

        def group(g, carry):
            cps = []
            for k in range(_G):
                cps.append(
                    pltpu.async_copy(
                        table_hbm.at[idx_v.at[g * _G + k]],
                        rows_v.at[pl.ds(k * _CK, _CK)],
                        gsem,
                    )
                )
            for cp in cps:
                cp.wait()
            pltpu.sync_copy(
                rows_v,
                out_hbm.at[pl.ds(base + g * (_G * _CK), _G * _CK)],
            )
            return carry

        lax.fori_loop(0, ng, group, 0)

    return sc_gather


def kernel(category, table, W, b):
    B, F = category.shape
    V, D = table.shape
    N = B * F

    tableT = table.T
    table2 = _make_transform(V, D)(tableT, W, b.reshape(1, D))
    VL = table2.shape[0] * 2
    table_lin = table2.reshape(VL, D)

    ch = N // (_NW * _CK)
    idx3 = category.T.reshape(_NW, ch, _CK)

    staging = _make_sc_gather(VL, D, N)(table_lin, idx3)
    out3 = staging.reshape(F, B, D)
    return jnp.transpose(out3, (1, 0, 2))


# speedup vs baseline: 3.0059x; 1.2822x over previous
import functools

import jax
import jax.numpy as jnp
from jax import lax
from jax.experimental import pallas as pl
from jax.experimental.pallas import tpu as pltpu
from jax.experimental.pallas import tpu_sc as plsc

_NC = 2
_NS = 16
_NW = _NC * _NS
_CK = 128
_SB = 8192
_SG = 2048


def _make_transform(V, D):
    nblk = -(-V // _SB)
    HB = _SB // 2

    def body(x_ref, w_ref, b_ref, o_ref):
        dn = (((0,), (1,)), ((), ()))
        x = x_ref[...]
        y1 = lax.dot_general(x[:, :HB], w_ref[...], dn,
                             preferred_element_type=jnp.float32)
        y2 = lax.dot_general(x[:, HB:], w_ref[...], dn,
                             preferred_element_type=jnp.float32)
        y1 = jnp.maximum(y1 + b_ref[...], 0.0)
        y2 = jnp.maximum(y2 + b_ref[...], 0.0)
        o_ref[...] = jnp.concatenate([y1, y2], axis=1)

    return pl.pallas_call(
        body,
        grid=(nblk,),
        in_specs=[
            pl.BlockSpec((D, _SB), lambda i: (0, i)),
            pl.BlockSpec((D, D), lambda i: (0, 0)),
            pl.BlockSpec((1, D), lambda i: (0, 0)),
        ],
        out_specs=pl.BlockSpec((HB, 2 * D), lambda i: (i, 0)),
        out_shape=jax.ShapeDtypeStruct((nblk * HB, 2 * D), jnp.float32),
    )


def _make_sc_gather(VL, D, B, F):
    bpw = B // _NW
    kpw = bpw // _CK
    mesh = plsc.VectorSubcoreMesh(core_axis_name="c", subcore_axis_name="s")

    @functools.partial(
        pl.kernel,
        out_type=jax.ShapeDtypeStruct((F, B // 2, 2 * D), jnp.float32),
        mesh=mesh,
        scratch_types=[
            pltpu.VMEM((F, kpw, _CK), jnp.int32),
            pltpu.VMEM((2, kpw, _CK, D), jnp.float32),
            pltpu.SemaphoreType.DMA,
            pltpu.SemaphoreType.DMA,
        ],
        compiler_params=pltpu.CompilerParams(
            use_tc_tiling_on_sc=False, needs_layout_passes=False,
        ),
    )
    def sc_gather(table_hbm, idx_hbm, out_hbm, idx_v, bufs, gsem, osem):
        cid = lax.axis_index("c")
        sid = lax.axis_index("s")
        wid = cid * _NS + sid
        b0 = wid * bpw
        pltpu.sync_copy(idx_hbm.at[:, pl.ds(wid * kpw, kpw), :], idx_v)

        def xform(f, carry):
            for k in range(kpw):
                for i in range(_CK // 16):
                    t = idx_v[f, k, pl.ds(i * 16, 16)]
                    a = t & (-_SB)
                    m = t & (_SB // 2 - 1)
                    h = (t >> 12) & 1
                    idx_v[f, k, pl.ds(i * 16, 16)] = a + m + m + h
            return carry

        lax.fori_loop(0, F, xform, 0)

        def fire(f, p):
            for k in range(kpw):
                pltpu.async_copy(
                    table_hbm.at[idx_v.at[f, k]],
                    bufs.at[p, k],
                    gsem,
                )

        def out_dst(f, k):
            bk = b0 + k * _CK
            row0 = ((bk >> 12) << 11) + (bk & (_SG - 1))
            lane0 = ((bk >> 11) & 1) * D
            return out_hbm.at[f, pl.ds(row0, _CK), pl.ds(lane0, D)]

        fire(0, 0)

        def field(f, carry):
            p = f & 1

            @pl.when(f > 0)
            def _():
                for k in range(kpw):
                    pltpu.make_async_copy(
                        bufs.at[1 - p, k], out_dst(f - 1, k), osem,
                    ).wait()

            @pl.when(f + 1 < F)
            def _():
                fire(f + 1, 1 - p)

            for k in range(kpw):
                pltpu.make_async_copy(
                    table_hbm.at[idx_v.at[f, k]],
                    bufs.at[p, k],
                    gsem,
                ).wait()

            for k in range(kpw):
                pltpu.async_copy(bufs.at[p, k], out_dst(f, k), osem)
            return carry

        lax.fori_loop(0, F, field, 0)

        for k in range(kpw):
            pltpu.make_async_copy(
                bufs.at[(F - 1) & 1, k], out_dst(F - 1, k), osem,
            ).wait()

    return sc_gather


def _make_unpack(B, F, D):
    def body(x_ref, i_ref, o_ref):
        x = x_ref[0]
        dn = (((1,), (1,)), ((), ()))
        ye = lax.dot_general(i_ref[...], x[:, :D], dn,
                             preferred_element_type=jnp.float32)
        yo = lax.dot_general(i_ref[...], x[:, D:], dn,
                             preferred_element_type=jnp.float32)
        o_ref[0, :, :_SG] = ye
        o_ref[0, :, _SG:] = yo

    return pl.pallas_call(
        body,
        grid=(F, B // (2 * _SG)),
        in_specs=[
            pl.BlockSpec((1, _SG, 2 * D), lambda f, q: (f, q, 0)),
            pl.BlockSpec((D, D), lambda f, q: (0, 0)),
        ],
        out_specs=pl.BlockSpec((1, D, 2 * _SG), lambda f, q: (f, 0, q)),
        out_shape=jax.ShapeDtypeStruct((F, D, B), jnp.float32),
    )


def kernel(category, table, W, b):
    B, F = category.shape
    V, D = table.shape

    tableT = table.T
    table2 = _make_transform(V, D)(tableT, W, b.reshape(1, D))
    VL = table2.shape[0] * 2
    table_lin = table2.reshape(VL, D)

    idxH = category.T.reshape(F, B // _CK, _CK)

    staging = _make_sc_gather(VL, D, B, F)(table_lin, idxH)
    eye = jnp.eye(D, dtype=jnp.float32)
    out3 = _make_unpack(B, F, D)(staging, eye)
    return jnp.transpose(out3, (2, 0, 1))


# speedup vs baseline: 3.4785x; 1.1572x over previous
import functools

import jax
import jax.numpy as jnp
from jax import lax
from jax.experimental import pallas as pl
from jax.experimental.pallas import tpu as pltpu
from jax.experimental.pallas import tpu_sc as plsc

_NC = 2
_NS = 16
_NW = _NC * _NS
_CK = 128
_SB = 8192
_SG = 2048


def _make_transform(V, D):
    nstep = -(-V // (2 * _SB))
    HB = _SB // 2

    def body(x_ref, w_ref, b_ref, o_ref):
        dn = (((0,), (1,)), ((), ()))
        x = x_ref[...]
        for s in range(2):
            ys = []
            for hh in range(2):
                lo = s * _SB + hh * HB
                y = lax.dot_general(x[:, lo:lo + HB], w_ref[...], dn,
                                    preferred_element_type=jnp.float32)
                ys.append(jnp.maximum(y + b_ref[...], 0.0))
            o_ref[pl.ds(s * HB, HB), :] = jnp.concatenate(ys, axis=1)

    return pl.pallas_call(
        body,
        grid=(nstep,),
        in_specs=[
            pl.BlockSpec((D, 2 * _SB), lambda i: (0, i)),
            pl.BlockSpec((D, D), lambda i: (0, 0)),
            pl.BlockSpec((1, D), lambda i: (0, 0)),
        ],
        out_specs=pl.BlockSpec((_SB, 2 * D), lambda i: (i, 0)),
        out_shape=jax.ShapeDtypeStruct((nstep * _SB, 2 * D), jnp.float32),
    )


def _make_sc_gather(VL, D, B, F):
    bpw = B // _NW
    kpw = bpw // _CK
    mesh = plsc.VectorSubcoreMesh(core_axis_name="c", subcore_axis_name="s")

    @functools.partial(
        pl.kernel,
        out_type=jax.ShapeDtypeStruct((F, B // 2, 2 * D), jnp.float32),
        mesh=mesh,
        scratch_types=[
            pltpu.VMEM((F, kpw, _CK), jnp.int32),
            pltpu.VMEM((2, kpw, _CK, D), jnp.float32),
            pltpu.SemaphoreType.DMA,
            pltpu.SemaphoreType.DMA,
        ],
        compiler_params=pltpu.CompilerParams(
            use_tc_tiling_on_sc=False, needs_layout_passes=False,
        ),
    )
    def sc_gather(table_hbm, idx_hbm, out_hbm, idx_v, bufs, gsem, osem):
        cid = lax.axis_index("c")
        sid = lax.axis_index("s")
        wid = cid * _NS + sid
        b0 = wid * bpw
        pltpu.sync_copy(idx_hbm.at[:, pl.ds(wid * kpw, kpw), :], idx_v)

        def xform(f, carry):
            for k in range(kpw):
                for i in range(_CK // 16):
                    t = idx_v[f, k, pl.ds(i * 16, 16)]
                    a = t & (-_SB)
                    m = t & (_SB // 2 - 1)
                    h = (t >> 12) & 1
                    idx_v[f, k, pl.ds(i * 16, 16)] = a + m + m + h
            return carry

        lax.fori_loop(0, F, xform, 0)

        def fire(f, p):
            for k in range(kpw):
                pltpu.async_copy(
                    table_hbm.at[idx_v.at[f, k]],
                    bufs.at[p, k],
                    gsem,
                )

        def out_dst(f, k):
            bk = b0 + k * _CK
            row0 = ((bk >> 12) << 11) + (bk & (_SG - 1))
            lane0 = ((bk >> 11) & 1) * D
            return out_hbm.at[f, pl.ds(row0, _CK), pl.ds(lane0, D)]

        fire(0, 0)

        def field(f, carry):
            p = f & 1

            @pl.when(f > 0)
            def _():
                for k in range(kpw):
                    pltpu.make_async_copy(
                        bufs.at[1 - p, k], out_dst(f - 1, k), osem,
                    ).wait()

            @pl.when(f + 1 < F)
            def _():
                fire(f + 1, 1 - p)

            for k in range(kpw):
                pltpu.make_async_copy(
                    table_hbm.at[idx_v.at[f, k]],
                    bufs.at[p, k],
                    gsem,
                ).wait()

            for k in range(kpw):
                pltpu.async_copy(bufs.at[p, k], out_dst(f, k), osem)
            return carry

        lax.fori_loop(0, F, field, 0)

        for k in range(kpw):
            pltpu.make_async_copy(
                bufs.at[(F - 1) & 1, k], out_dst(F - 1, k), osem,
            ).wait()

    return sc_gather


def _make_unpack(B, F, D):
    def body(x_ref, i_ref, o_ref):
        dn = (((1,), (1,)), ((), ()))
        for g in range(2):
            x = x_ref[0, pl.ds(g * _SG, _SG), :]
            for hh in range(2):
                y = lax.dot_general(i_ref[...], x[:, hh * D:(hh + 1) * D],
                                    dn, preferred_element_type=jnp.float32)
                o_ref[0, :, pl.ds((2 * g + hh) * _SG, _SG)] = y

    return pl.pallas_call(
        body,
        grid=(F, B // (4 * _SG)),
        in_specs=[
            pl.BlockSpec((1, 2 * _SG, 2 * D), lambda f, q: (f, q, 0)),
            pl.BlockSpec((D, D), lambda f, q: (0, 0)),
        ],
        out_specs=pl.BlockSpec((1, D, 4 * _SG), lambda f, q: (f, 0, q)),
        out_shape=jax.ShapeDtypeStruct((F, D, B), jnp.float32),
    )


def kernel(category, table, W, b):
    B, F = category.shape
    V, D = table.shape

    tableT = table.T
    table2 = _make_transform(V, D)(tableT, W, b.reshape(1, D))
    VL = table2.shape[0] * 2
    table_lin = table2.reshape(VL, D)

    idxH = category.T.reshape(F, B // _CK, _CK)

    staging = _make_sc_gather(VL, D, B, F)(table_lin, idxH)
    eye = jnp.eye(D, dtype=jnp.float32)
    out3 = _make_unpack(B, F, D)(staging, eye)
    return jnp.transpose(out3, (2, 0, 1))


# speedup vs baseline: 3.7532x; 1.0790x over previous
import functools

import jax
import jax.numpy as jnp
from jax import lax
from jax.experimental import pallas as pl
from jax.experimental.pallas import tpu as pltpu
from jax.experimental.pallas import tpu_sc as plsc

_NC = 2
_NS = 16
_NW = _NC * _NS
_CK = 128
_SB = 8192
_SG = 2048


def _make_transform(V, D):
    nsb = 4
    nstep = -(-V // (nsb * _SB))
    HB = _SB // 2

    def body(x_ref, w_ref, b_ref, o_ref):
        dn = (((0,), (1,)), ((), ()))
        x = x_ref[...]
        for s in range(nsb):
            ys = []
            for hh in range(2):
                lo = s * _SB + hh * HB
                y = lax.dot_general(x[:, lo:lo + HB], w_ref[...], dn,
                                    preferred_element_type=jnp.float32)
                ys.append(jnp.maximum(y + b_ref[...], 0.0))
            o_ref[pl.ds(s * HB, HB), :] = jnp.concatenate(ys, axis=1)

    return pl.pallas_call(
        body,
        grid=(nstep,),
        in_specs=[
            pl.BlockSpec((D, nsb * _SB), lambda i: (0, i)),
            pl.BlockSpec((D, D), lambda i: (0, 0)),
            pl.BlockSpec((1, D), lambda i: (0, 0)),
        ],
        out_specs=pl.BlockSpec((nsb * HB, 2 * D), lambda i: (i, 0)),
        out_shape=jax.ShapeDtypeStruct((nstep * nsb * HB, 2 * D), jnp.float32),
    )


def _make_sc_gather(VL, D, B, F):
    bpw = B // _NW
    kpw = bpw // _CK
    mesh = plsc.VectorSubcoreMesh(core_axis_name="c", subcore_axis_name="s")

    @functools.partial(
        pl.kernel,
        out_type=jax.ShapeDtypeStruct((F, B // 2, 2 * D), jnp.float32),
        mesh=mesh,
        scratch_types=[
            pltpu.VMEM((F, kpw, _CK), jnp.int32),
            pltpu.VMEM((2, kpw, _CK, D), jnp.float32),
            pltpu.SemaphoreType.DMA,
            pltpu.SemaphoreType.DMA,
        ],
        compiler_params=pltpu.CompilerParams(
            use_tc_tiling_on_sc=False, needs_layout_passes=False,
        ),
    )
    def sc_gather(table_hbm, idx_hbm, out_hbm, idx_v, bufs, gsem, osem):
        cid = lax.axis_index("c")
        sid = lax.axis_index("s")
        wid = cid * _NS + sid
        b0 = wid * bpw
        pltpu.sync_copy(idx_hbm.at[:, pl.ds(wid * kpw, kpw), :], idx_v)

        def xform(f, carry):
            for k in range(kpw):
                for i in range(_CK // 16):
                    t = idx_v[f, k, pl.ds(i * 16, 16)]
                    a = t & (-_SB)
                    m = t & (_SB // 2 - 1)
                    h = (t >> 12) & 1
                    idx_v[f, k, pl.ds(i * 16, 16)] = a + m + m + h
            return carry

        lax.fori_loop(0, F, xform, 0)

        def fire(f, p):
            for k in range(kpw):
                pltpu.async_copy(
                    table_hbm.at[idx_v.at[f, k]],
                    bufs.at[p, k],
                    gsem,
                )

        def out_dst(f, k):
            bk = b0 + k * _CK
            row0 = ((bk >> 12) << 11) + (bk & (_SG - 1))
            lane0 = ((bk >> 11) & 1) * D
            return out_hbm.at[f, pl.ds(row0, _CK), pl.ds(lane0, D)]

        fire(0, 0)

        def field(f, carry):
            p = f & 1

            @pl.when(f > 0)
            def _():
                for k in range(kpw):
                    pltpu.make_async_copy(
                        bufs.at[1 - p, k], out_dst(f - 1, k), osem,
                    ).wait()

            @pl.when(f + 1 < F)
            def _():
                fire(f + 1, 1 - p)

            for k in range(kpw):
                pltpu.make_async_copy(
                    table_hbm.at[idx_v.at[f, k]],
                    bufs.at[p, k],
                    gsem,
                ).wait()

            for k in range(kpw):
                pltpu.async_copy(bufs.at[p, k], out_dst(f, k), osem)
            return carry

        lax.fori_loop(0, F, field, 0)

        for k in range(kpw):
            pltpu.make_async_copy(
                bufs.at[(F - 1) & 1, k], out_dst(F - 1, k), osem,
            ).wait()

    return sc_gather


def _make_unpack(B, F, D):
    ng = B // (2 * _SG)

    def body(x_ref, i_ref, o_ref):
        dn = (((1,), (1,)), ((), ()))
        for g in range(ng):
            x = x_ref[0, pl.ds(g * _SG, _SG), :]
            for hh in range(2):
                y = lax.dot_general(i_ref[...], x[:, hh * D:(hh + 1) * D],
                                    dn, preferred_element_type=jnp.float32)
                o_ref[0, :, pl.ds((2 * g + hh) * _SG, _SG)] = y

    return pl.pallas_call(
        body,
        grid=(F,),
        in_specs=[
            pl.BlockSpec((1, ng * _SG, 2 * D), lambda f: (f, 0, 0)),
            pl.BlockSpec((D, D), lambda f: (0, 0)),
        ],
        out_specs=pl.BlockSpec((1, D, B), lambda f: (f, 0, 0)),
        out_shape=jax.ShapeDtypeStruct((F, D, B), jnp.float32),
    )


def kernel(category, table, W, b):
    B, F = category.shape
    V, D = table.shape

    tableT = table.T
    table2 = _make_transform(V, D)(tableT, W, b.reshape(1, D))
    VL = table2.shape[0] * 2
    table_lin = table2.reshape(VL, D)

    idxH = category.T.reshape(F, B // _CK, _CK)

    staging = _make_sc_gather(VL, D, B, F)(table_lin, idxH)
    eye = jnp.eye(D, dtype=jnp.float32)
    out3 = _make_unpack(B, F, D)(staging, eye)
    return jnp.transpose(out3, (2, 0, 1))


# speedup vs baseline: 4.1587x; 1.1081x over previous
import functools

import jax
import jax.numpy as jnp
from jax import lax
from jax.experimental import pallas as pl
from jax.experimental.pallas import tpu as pltpu
from jax.experimental.pallas import tpu_sc as plsc

_NC = 2
_NS = 16
_NW = _NC * _NS
_CK = 128
_SB = 8192
_SG = 2048


def _make_transform(V, D):
    nsb = 4
    nstep = -(-V // (nsb * _SB))
    HB = _SB // 2

    def body(x_ref, w_ref, b_ref, o_ref):
        dn = (((0,), (1,)), ((), ()))
        x = x_ref[...].astype(jnp.bfloat16)
        w = w_ref[...].astype(jnp.bfloat16)
        for s in range(nsb):
            ys = []
            for hh in range(2):
                lo = s * _SB + hh * HB
                y = lax.dot_general(x[:, lo:lo + HB], w, dn,
                                    preferred_element_type=jnp.float32)
                ys.append(jnp.maximum(y + b_ref[...], 0.0))
            o_ref[pl.ds(s * HB, HB), :] = jnp.concatenate(ys, axis=1)

    return pl.pallas_call(
        body,
        grid=(nstep,),
        in_specs=[
            pl.BlockSpec((D, nsb * _SB), lambda i: (0, i)),
            pl.BlockSpec((D, D), lambda i: (0, 0)),
            pl.BlockSpec((1, D), lambda i: (0, 0)),
        ],
        out_specs=pl.BlockSpec((nsb * HB, 2 * D), lambda i: (i, 0)),
        out_shape=jax.ShapeDtypeStruct((nstep * nsb * HB, 2 * D), jnp.float32),
    )


def _make_sc_gather(VL, D, B, F):
    bpw = B // _NW
    kpw = bpw // _CK
    mesh = plsc.VectorSubcoreMesh(core_axis_name="c", subcore_axis_name="s")

    @functools.partial(
        pl.kernel,
        out_type=jax.ShapeDtypeStruct((F, B // 2, 2 * D), jnp.float32),
        mesh=mesh,
        scratch_types=[
            pltpu.VMEM((F, kpw, _CK), jnp.int32),
            pltpu.VMEM((2, kpw, _CK, D), jnp.float32),
            pltpu.SemaphoreType.DMA,
            pltpu.SemaphoreType.DMA,
        ],
        compiler_params=pltpu.CompilerParams(
            use_tc_tiling_on_sc=False, needs_layout_passes=False,
        ),
    )
    def sc_gather(table_hbm, idx_hbm, out_hbm, idx_v, bufs, gsem, osem):
        cid = lax.axis_index("c")
        sid = lax.axis_index("s")
        wid = cid * _NS + sid
        b0 = wid * bpw
        pltpu.sync_copy(idx_hbm.at[:, pl.ds(wid * kpw, kpw), :], idx_v)

        def xform(f, carry):
            for k in range(kpw):
                for i in range(_CK // 16):
                    t = idx_v[f, k, pl.ds(i * 16, 16)]
                    a = t & (-_SB)
                    m = t & (_SB // 2 - 1)
                    h = (t >> 12) & 1
                    idx_v[f, k, pl.ds(i * 16, 16)] = a + m + m + h
            return carry

        lax.fori_loop(0, F, xform, 0)

        def fire(f, p):
            for k in range(kpw):
                pltpu.async_copy(
                    table_hbm.at[idx_v.at[f, k]],
                    bufs.at[p, k],
                    gsem,
                )

        def out_dst(f, k):
            bk = b0 + k * _CK
            row0 = ((bk >> 12) << 11) + (bk & (_SG - 1))
            lane0 = ((bk >> 11) & 1) * D
            return out_hbm.at[f, pl.ds(row0, _CK), pl.ds(lane0, D)]

        fire(0, 0)

        def field(f, carry):
            p = f & 1

            @pl.when(f > 0)
            def _():
                for k in range(kpw):
                    pltpu.make_async_copy(
                        bufs.at[1 - p, k], out_dst(f - 1, k), osem,
                    ).wait()

            @pl.when(f + 1 < F)
            def _():
                fire(f + 1, 1 - p)

            for k in range(kpw):
                pltpu.make_async_copy(
                    table_hbm.at[idx_v.at[f, k]],
                    bufs.at[p, k],
                    gsem,
                ).wait()

            for k in range(kpw):
                pltpu.async_copy(bufs.at[p, k], out_dst(f, k), osem)
            return carry

        lax.fori_loop(0, F, field, 0)

        for k in range(kpw):
            pltpu.make_async_copy(
                bufs.at[(F - 1) & 1, k], out_dst(F - 1, k), osem,
            ).wait()

    return sc_gather


def _make_unpack(B, F, D):
    ng = B // (2 * _SG)

    def body(x_ref, i_ref, o_ref):
        dn = (((1,), (1,)), ((), ()))
        for g in range(ng):
            x = x_ref[0, pl.ds(g * _SG, _SG), :]
            for hh in range(2):
                y = lax.dot_general(i_ref[...], x[:, hh * D:(hh + 1) * D],
                                    dn, preferred_element_type=jnp.float32)
                o_ref[0, :, pl.ds((2 * g + hh) * _SG, _SG)] = y

    return pl.pallas_call(
        body,
        grid=(F,),
        in_specs=[
            pl.BlockSpec((1, ng * _SG, 2 * D), lambda f: (f, 0, 0)),
            pl.BlockSpec((D, D), lambda f: (0, 0)),
        ],
        out_specs=pl.BlockSpec((1, D, B), lambda f: (f, 0, 0)),
        out_shape=jax.ShapeDtypeStruct((F, D, B), jnp.float32),
    )


def kernel(category, table, W, b):
    B, F = category.shape
    V, D = table.shape

    tableT = table.T
    table2 = _make_transform(V, D)(tableT, W, b.reshape(1, D))
    VL = table2.shape[0] * 2
    table_lin = table2.reshape(VL, D)

    idxH = category.T.reshape(F, B // _CK, _CK)

    staging = _make_sc_gather(VL, D, B, F)(table_lin, idxH)
    eye = jnp.eye(D, dtype=jnp.float32)
    out3 = _make_unpack(B, F, D)(staging, eye)
    return jnp.transpose(out3, (2, 0, 1))
